# Initial kernel scaffold; baseline (speedup 1.0000x reference)
import jax
import jax.numpy as jnp
from jax.experimental import pallas as pl


def kernel(vecs, c_sum, c_count):
    raise NotImplementedError("write your pallas kernel here")


# speedup vs baseline: 1.2610x; 1.2610x over previous
import functools

import jax
import jax.numpy as jnp
from jax import lax
from jax.experimental import pallas as pl
from jax.experimental.pallas import tpu as pltpu
from jax.experimental.pallas import tpu_sc as plsc

N_CODE = 1024
D_K = 256


_RB = 1024


def _dist_body(v_ref, c_ref, cn_ref, z_ref, lsum_ref):
    i = pl.program_id(0)

    @pl.when(i == 0)
    def _init():
        lsum_ref[0, 0] = 0.0

    v = v_ref[...]
    c = c_ref[...]
    s = lax.dot_general(v, c, (((1,), (1,)), ((), ())),
                        preferred_element_type=jnp.float32)
    vn = jnp.sum(v * v, axis=1, keepdims=True)
    diffs = (vn + (-2.0) * s) + cn_ref[...][None, :]
    m = jnp.min(diffs, axis=1, keepdims=True)
    ids = lax.broadcasted_iota(jnp.int32, (_RB, N_CODE), 1)
    z = jnp.min(jnp.where(diffs == m, ids, N_CODE), axis=1)
    z_ref[...] = z.astype(jnp.int32)
    lsum_ref[0, 0] += jnp.sum(jnp.maximum(m, 0.0))


def _distances_argmin(v2, c, cn):
    n = v2.shape[0]
    grid = (n // _RB,)
    z, lsum = pl.pallas_call(
        _dist_body,
        grid=grid,
        in_specs=[
            pl.BlockSpec((_RB, D_K), lambda i: (i, 0)),
            pl.BlockSpec((N_CODE, D_K), lambda i: (0, 0)),
            pl.BlockSpec((N_CODE,), lambda i: (0,)),
        ],
        out_specs=[
            pl.BlockSpec((_RB,), lambda i: (i,)),
            pl.BlockSpec(memory_space=pltpu.SMEM),
        ],
        out_shape=[
            jax.ShapeDtypeStruct((n,), jnp.int32),
            jax.ShapeDtypeStruct((1, 1), jnp.float32),
        ],
    )(v2, c, cn)
    return z, lsum


_NC = 2
_NS = 16
_NW = _NC * _NS
_CHUNK = 128


def _sc_gather(table, idx3, n_rows):
    b_per_w = n_rows // _NW
    n_chunk = b_per_w // _CHUNK
    mesh = plsc.VectorSubcoreMesh(core_axis_name="c", subcore_axis_name="s")

    @functools.partial(
        pl.kernel,
        mesh=mesh,
        out_type=jax.ShapeDtypeStruct((n_rows, D_K), jnp.float32),
        scratch_types=[
            pltpu.VMEM((n_chunk, _CHUNK), jnp.int32),
            pltpu.VMEM((_CHUNK, D_K), jnp.float32),
            pltpu.VMEM((_CHUNK, D_K), jnp.float32),
            pltpu.SemaphoreType.DMA,
            pltpu.SemaphoreType.DMA,
        ],
    )
    def gather_k(table_hbm, idx_hbm, out_hbm, idx_v, buf0, buf1, sem0, sem1):
        wid = lax.axis_index("s") * _NC + lax.axis_index("c")
        base = wid * b_per_w
        pltpu.sync_copy(idx_hbm.at[wid], idx_v)
        bufs = (buf0, buf1)
        sems = (sem0, sem1)
        cp = pltpu.async_copy(table_hbm.at[idx_v.at[0]], bufs[0], sems[0])
        for j in range(n_chunk):
            cur = cp
            if j + 1 < n_chunk:
                cp = pltpu.async_copy(
                    table_hbm.at[idx_v.at[j + 1]], bufs[(j + 1) % 2],
                    sems[(j + 1) % 2])
            cur.wait()
            pltpu.sync_copy(bufs[j % 2],
                            out_hbm.at[pl.ds(base + j * _CHUNK, _CHUNK)])

    return gather_k(table, idx3)


def kernel(vecs, c_sum, c_count):
    b, r, cdim, k = vecs.shape
    n = b * r * cdim
    v2 = vecs.astype(jnp.float32).reshape(n, k)
    c = jnp.divide(c_sum, jnp.clip(jnp.expand_dims(c_count, -1), 0.01))
    c = c.astype(jnp.float32)
    cn = jnp.einsum('sk->s', jnp.square(c))

    z_flat, lsum = _distances_argmin(v2, c, cn)
    l_commit = lsum[0, 0] / (b * r)

    idx3 = z_flat.reshape(_NW, n // (_NW * _CHUNK), _CHUNK)
    vecs_hat = _sc_gather(c, idx3, n).reshape(b, r, cdim, k).astype(vecs.dtype)

    z = z_flat.reshape(b, r, cdim)
    l_codebook = jnp.zeros((), jnp.float32)
    return vecs_hat, z, l_commit, l_codebook
